# Initial kernel scaffold; baseline (speedup 1.0000x reference)
import jax
import jax.numpy as jnp
from jax.experimental import pallas as pl


def kernel(x, edge_index, W_self, W_neigh, b):
    raise NotImplementedError("write your pallas kernel here")


# speedup vs baseline: 5.9352x; 5.9352x over previous
import functools

import jax
import jax.numpy as jnp
from jax import lax
from jax.experimental import pallas as pl
from jax.experimental.pallas import tpu as pltpu
from jax.experimental.pallas import tpu_sc as plsc

NC = 2
NS = 16
DEGW = 16
HALF = 64
WID = HALF + DEGW
BATCH = 128
STAGE = 8
N_PAD = 10240


def _sc_scatter(src3d, dst3d, xa):
    _, chunks, batch = src3d.shape
    n = xa.shape[0] // NC
    rows_per_sub = N_PAD // NS
    zrows = 128
    zcopies = rows_per_sub // zrows

    mesh = plsc.VectorSubcoreMesh(core_axis_name="c", subcore_axis_name="s")

    @functools.partial(
        pl.kernel,
        out_type=jax.ShapeDtypeStruct((NC * N_PAD, WID), jnp.float32),
        mesh=mesh,
        scratch_types=[
            pltpu.VMEM((STAGE, batch), jnp.int32),
            pltpu.VMEM((STAGE, batch), jnp.int32),
            pltpu.VMEM((batch, WID), jnp.float32),
            pltpu.VMEM_SHARED((N_PAD, WID), jnp.float32),
            pltpu.SemaphoreType.DMA,
        ],
        compiler_params=pltpu.CompilerParams(use_tc_tiling_on_sc=False),
    )
    def sc_kernel(src_hbm, dst_hbm, xa_hbm, acc_out,
                  src_v, dst_v, rows_v, acc_sh, sem):
        cid = lax.axis_index("c")
        sid = lax.axis_index("s")

        zseg = jnp.zeros((16,), jnp.float32)

        def zero_row(i, carry):
            for j in range(WID // 16):
                rows_v[i, pl.ds(j * 16, 16)] = zseg
            return carry

        lax.fori_loop(0, zrows, zero_row, 0)

        base = sid * rows_per_sub
        for k in range(zcopies):
            pltpu.sync_copy(rows_v, acc_sh.at[pl.ds(base + k * zrows, zrows)])
        plsc.subcore_barrier()

        table_off = cid * n

        def process_stage(h, carry):
            pltpu.sync_copy(src_hbm.at[sid, pl.ds(h * STAGE, STAGE)], src_v)
            pltpu.sync_copy(dst_hbm.at[sid, pl.ds(h * STAGE, STAGE)], dst_v)

            def add_off(i, c2):
                for j in range(batch // 16):
                    sl = pl.ds(j * 16, 16)
                    src_v[i, sl] = src_v[i, sl] + table_off
                return c2

            lax.fori_loop(0, STAGE, add_off, 0)

            def step(j, c3):
                pltpu.async_copy(xa_hbm.at[src_v.at[j]], rows_v, sem).wait()
                pltpu.sync_copy(rows_v, acc_sh.at[dst_v.at[j]], add=True)
                return c3

            lax.fori_loop(0, STAGE, step, 0)
            return carry

        lax.fori_loop(0, chunks // STAGE, process_stage, 0)
        plsc.subcore_barrier()

        out_base = cid * N_PAD + base
        for k in range(zcopies):
            pltpu.sync_copy(acc_sh.at[pl.ds(base + k * zrows, zrows)], rows_v)
            pltpu.sync_copy(rows_v,
                            acc_out.at[pl.ds(out_base + k * zrows, zrows)])

    return sc_kernel(src3d, dst3d, xa)


def _tc_combine(x, agg, deg, w_self, w_neigh, b2d):
    n, d = x.shape
    blk = 1000

    def body(x_ref, a_ref, dg_ref, ws_ref, wn_ref, b_ref, o_ref):
        deg_col = jnp.maximum(dg_ref[:, 0:1], 1.0)
        agg_n = a_ref[...] / deg_col
        acc = jnp.dot(x_ref[...], ws_ref[...],
                      preferred_element_type=jnp.float32)
        acc = acc + jnp.dot(agg_n, wn_ref[...],
                            preferred_element_type=jnp.float32)
        o_ref[...] = jnp.maximum(acc + b_ref[...], 0.0)

    return pl.pallas_call(
        body,
        grid=(n // blk,),
        in_specs=[
            pl.BlockSpec((blk, d), lambda i: (i, 0)),
            pl.BlockSpec((blk, d), lambda i: (i, 0)),
            pl.BlockSpec((blk, DEGW), lambda i: (i, 0)),
            pl.BlockSpec((d, d), lambda i: (0, 0)),
            pl.BlockSpec((d, d), lambda i: (0, 0)),
            pl.BlockSpec((1, d), lambda i: (0, 0)),
        ],
        out_specs=pl.BlockSpec((blk, d), lambda i: (i, 0)),
        out_shape=jax.ShapeDtypeStruct((n, d), jnp.float32),
    )(x, agg, deg, w_self, w_neigh, b2d)


def kernel(x, edge_index, W_self, W_neigh, b):
    n, d = x.shape
    e = edge_index.shape[1]
    e_pad = NS * 160 * BATCH
    pad = e_pad - e

    onecol = jnp.concatenate(
        [jnp.ones((n, 1), jnp.float32),
         jnp.zeros((n, DEGW - 1), jnp.float32)], axis=1)
    xa = jnp.concatenate(
        [jnp.concatenate([x[:, :HALF], onecol], axis=1),
         jnp.concatenate([x[:, HALF:], onecol], axis=1)], axis=0)

    ar = jnp.arange(pad, dtype=jnp.int32)
    src = jnp.concatenate([edge_index[0], ar % n])
    dst = jnp.concatenate([edge_index[1], n + ar % (N_PAD - n)])
    src3d = src.reshape(NS, e_pad // (NS * BATCH), BATCH)
    dst3d = dst.reshape(NS, e_pad // (NS * BATCH), BATCH)

    acc = _sc_scatter(src3d, dst3d, xa)
    agg = jnp.concatenate([acc[:n, :HALF],
                           acc[N_PAD:N_PAD + n, :HALF]], axis=1)
    deg = acc[:n, HALF:]
    return _tc_combine(x, agg, deg, W_self, W_neigh, b.reshape(1, d))


# speedup vs baseline: 8.3396x; 1.4051x over previous
import functools

import jax
import jax.numpy as jnp
from jax import lax
from jax.experimental import pallas as pl
from jax.experimental.pallas import tpu as pltpu
from jax.experimental.pallas import tpu_sc as plsc

NC = 2
NS = 16
DEGW = 16
HALF = 64
WID = HALF + DEGW
BATCH = 128
STAGE = 8
N_PAD = 10240


def _sc_scatter(src3d, dst3d, xa):
    _, chunks, batch = src3d.shape
    n = xa.shape[0] // NC
    rows_per_sub = N_PAD // NS
    zrows = 128
    zcopies = rows_per_sub // zrows

    mesh = plsc.VectorSubcoreMesh(core_axis_name="c", subcore_axis_name="s")

    @functools.partial(
        pl.kernel,
        out_type=jax.ShapeDtypeStruct((NC, N_PAD, WID), jnp.float32),
        mesh=mesh,
        scratch_types=[
            pltpu.VMEM((STAGE, batch), jnp.int32),
            pltpu.VMEM((STAGE, batch), jnp.int32),
            pltpu.VMEM((batch, WID), jnp.float32),
            pltpu.VMEM((batch, WID), jnp.float32),
            pltpu.VMEM_SHARED((N_PAD, WID), jnp.float32),
            pltpu.SemaphoreType.DMA,
            pltpu.SemaphoreType.DMA,
        ],
        compiler_params=pltpu.CompilerParams(use_tc_tiling_on_sc=False),
    )
    def sc_kernel(src_hbm, dst_hbm, xa_hbm, acc_out,
                  src_v, dst_v, rows_a, rows_b, acc_sh, sem_a, sem_b):
        cid = lax.axis_index("c")
        sid = lax.axis_index("s")

        zseg = jnp.zeros((16,), jnp.float32)

        def zero_row(i, carry):
            for j in range(WID // 16):
                rows_a[i, pl.ds(j * 16, 16)] = zseg
            return carry

        lax.fori_loop(0, zrows, zero_row, 0)

        base = sid * rows_per_sub
        for k in range(zcopies):
            pltpu.sync_copy(rows_a, acc_sh.at[pl.ds(base + k * zrows, zrows)])
        plsc.subcore_barrier()

        table_off = cid * n
        bufs = (rows_a, rows_b)
        sems = (sem_a, sem_b)

        def process_stage(h, carry):
            pltpu.sync_copy(src_hbm.at[sid, pl.ds(h * STAGE, STAGE)], src_v)
            pltpu.sync_copy(dst_hbm.at[sid, pl.ds(h * STAGE, STAGE)], dst_v)

            def add_off(i, c2):
                for j in range(batch // 16):
                    sl = pl.ds(j * 16, 16)
                    src_v[i, sl] = src_v[i, sl] + table_off
                return c2

            lax.fori_loop(0, STAGE, add_off, 0)

            waits = [None, None]
            waits[0] = pltpu.async_copy(
                xa_hbm.at[src_v.at[0]], bufs[0], sems[0])
            for j in range(STAGE):
                p = j % 2
                if j + 1 < STAGE:
                    waits[1 - p] = pltpu.async_copy(
                        xa_hbm.at[src_v.at[j + 1]], bufs[1 - p], sems[1 - p])
                waits[p].wait()
                pltpu.sync_copy(bufs[p], acc_sh.at[dst_v.at[j]], add=True)
            return carry

        lax.fori_loop(0, chunks // STAGE, process_stage, 0)
        plsc.subcore_barrier()

        for k in range(zcopies):
            pltpu.sync_copy(acc_sh.at[pl.ds(base + k * zrows, zrows)], rows_a)
            pltpu.sync_copy(
                rows_a, acc_out.at[cid, pl.ds(base + k * zrows, zrows)])

    return sc_kernel(src3d, dst3d, xa)


def _tc_combine(x, acc, w_self, w_neigh, b2d):
    n, d = x.shape
    blk = 1000

    def body(x_ref, al_ref, ar_ref, ws_ref, wn_ref, b_ref, o_ref):
        deg_col = jnp.maximum(al_ref[0, :, HALF:HALF + 1], 1.0)
        agg = jnp.concatenate([al_ref[0, :, :HALF], ar_ref[0, :, :HALF]],
                              axis=1) / deg_col
        acc_o = jnp.dot(x_ref[...], ws_ref[...],
                        preferred_element_type=jnp.float32)
        acc_o = acc_o + jnp.dot(agg, wn_ref[...],
                                preferred_element_type=jnp.float32)
        o_ref[...] = jnp.maximum(acc_o + b_ref[...], 0.0)

    return pl.pallas_call(
        body,
        grid=(n // blk,),
        in_specs=[
            pl.BlockSpec((blk, d), lambda i: (i, 0)),
            pl.BlockSpec((1, blk, WID), lambda i: (0, i, 0)),
            pl.BlockSpec((1, blk, WID), lambda i: (1, i, 0)),
            pl.BlockSpec((d, d), lambda i: (0, 0)),
            pl.BlockSpec((d, d), lambda i: (0, 0)),
            pl.BlockSpec((1, d), lambda i: (0, 0)),
        ],
        out_specs=pl.BlockSpec((blk, d), lambda i: (i, 0)),
        out_shape=jax.ShapeDtypeStruct((n, d), jnp.float32),
    )(x, acc, acc, w_self, w_neigh, b2d)


def kernel(x, edge_index, W_self, W_neigh, b):
    n, d = x.shape
    e = edge_index.shape[1]
    e_pad = NS * 160 * BATCH
    pad = e_pad - e

    onecol = jnp.concatenate(
        [jnp.ones((n, 1), jnp.float32),
         jnp.zeros((n, DEGW - 1), jnp.float32)], axis=1)
    xa = jnp.concatenate(
        [jnp.concatenate([x[:, :HALF], onecol], axis=1),
         jnp.concatenate([x[:, HALF:], onecol], axis=1)], axis=0)

    ar = jnp.arange(pad, dtype=jnp.int32)
    src = jnp.concatenate([edge_index[0], ar % n])
    dst = jnp.concatenate([edge_index[1], n + ar % (N_PAD - n)])
    src3d = src.reshape(NS, e_pad // (NS * BATCH), BATCH)
    dst3d = dst.reshape(NS, e_pad // (NS * BATCH), BATCH)

    acc = _sc_scatter(src3d, dst3d, xa)
    return _tc_combine(x, acc, W_self, W_neigh, b.reshape(1, d))
